# baseline (speedup 1.0000x reference)
import functools

import jax
import jax.numpy as jnp
from jax import lax
from jax.experimental import pallas as pl
from jax.experimental.pallas import tpu as pltpu
from jax.experimental.pallas import tpu_sc as plsc

_B = 512
_S = 14
_C = 30
_NTILES = 32
_HB = 256
_UPS = _B // _HB
_T_SPLIT = 6
_UNITS_TOTAL = _S * _S * _UPS
_SC_BASE = _T_SPLIT * _S * _UPS
_SC_UNITS = _UNITS_TOTAL - _SC_BASE
assert _SC_UNITS % _NTILES == 0
_SC_UNITS_PER_TILE = _SC_UNITS // _NTILES
_GROUPS = _HB // 16
_HALF = 0.5 * _S


def _sqrt16(x):
    i = plsc.bitcast(x, jnp.int32)
    i = jnp.int32(0x5F3759DF) - lax.shift_right_logical(i, 1)
    r = plsc.bitcast(i, jnp.float32)
    r = r * (1.5 - 0.5 * x * r * r)
    r = r * (1.5 - 0.5 * x * r * r)
    return x * r


def _sc_partials(pt, gt):
    mesh = plsc.VectorSubcoreMesh(core_axis_name="c", subcore_axis_name="s")

    @functools.partial(
        pl.kernel,
        mesh=mesh,
        out_type=jax.ShapeDtypeStruct((_NTILES, 16), jnp.float32),
        compiler_params=pltpu.CompilerParams(needs_layout_passes=False),
        scratch_types=[
            pltpu.VMEM((2, _C, _HB), jnp.float32),
            pltpu.VMEM((2, _C, _HB), jnp.float32),
            pltpu.VMEM((16,), jnp.float32),
            pltpu.SemaphoreType.DMA((2,)),
        ],
    )
    def body(p_hbm, g_hbm, out_hbm, pbufs, gbufs, acc, sems):
        wid = lax.axis_index("s") * 2 + lax.axis_index("c")
        acc[...] = jnp.zeros((16,), jnp.float32)

        def slices(uid):
            sid = uid // _UPS
            b0 = (uid - sid * _UPS) * _HB
            s1 = sid // _S
            s2 = sid - s1 * _S
            return (s1, s2, slice(None), pl.ds(b0, _HB))

        def start(k, par):
            uid = _SC_BASE + wid + k * _NTILES
            src = slices(uid)
            pltpu.async_copy(p_hbm.at[src], pbufs.at[par], sems.at[par])
            pltpu.async_copy(g_hbm.at[src], gbufs.at[par], sems.at[par])

        def wait_work(k, par):
            uid = _SC_BASE + wid + k * _NTILES
            src = slices(uid)
            pb = pbufs.at[par]
            gb = gbufs.at[par]
            pltpu.make_async_copy(p_hbm.at[src], pb, sems.at[par]).wait()
            pltpu.make_async_copy(g_hbm.at[src], gb, sems.at[par]).wait()

            def do_group(b0):
                def lp(c):
                    return pb[c, pl.ds(b0, 16)]

                def lg(c):
                    return gb[c, pl.ds(b0, 16)]

                px1, py1, pw1, ph1, pc1 = lp(0), lp(1), lp(2), lp(3), lp(4)
                px2, py2, pw2, ph2, pc2 = lp(5), lp(6), lp(7), lp(8), lp(9)
                gx, gy, gw, gh, g4 = lg(0), lg(1), lg(2), lg(3), lg(4)
                g9 = lg(9)

                cls = jnp.zeros((16,), jnp.float32)
                for c in range(10, _C):
                    dcv = lp(c) - lg(c)
                    cls = cls + dcv * dcv

                gltx = gx - _HALF * gw
                grbx = gx + _HALF * gw
                glty = gy - _HALF * gh
                grby = gy + _HALF * gh
                ag = (grbx - gltx) * (grby - glty)

                def iou_parts(px, py, pw, ph):
                    pltx = px - _HALF * pw
                    prbx = px + _HALF * pw
                    plty = py - _HALF * ph
                    prby = py + _HALF * ph
                    wx = jnp.maximum(
                        jnp.minimum(prbx, grbx) - jnp.maximum(pltx, gltx), 0.0)
                    wy = jnp.maximum(
                        jnp.minimum(prby, grby) - jnp.maximum(plty, glty), 0.0)
                    inter = wx * wy
                    ap = (prbx - pltx) * (prby - plty)
                    return inter, ap + ag - inter + 1e-10

                in1, de1 = iou_parts(px1, py1, pw1, ph1)
                in2, de2 = iou_parts(px2, py2, pw2, ph2)
                sel = in2 * de1 > in1 * de2
                rx = jnp.where(sel, px2, px1)
                ry = jnp.where(sel, py2, py1)
                rw = jnp.where(sel, pw2, pw1)
                rh = jnp.where(sel, ph2, ph1)
                rc = jnp.where(sel, pc2, pc1)
                ic = jnp.where(sel, pc1, pc2)
                miou = jnp.where(sel, in2, in1) / jnp.where(sel, de2, de1)

                dx = rx - gx
                dy = ry - gy
                coord = (dx * dx + dy * dy
                         + (rw + gw - 2.0 * _sqrt16(rw * gw))
                         + (rh + gh - 2.0 * _sqrt16(rh * gh)))
                dresp = rc - miou
                resp = dresp * dresp
                irr = ic * ic
                d4 = pc1 - g4
                d9 = pc2 - g9
                noobj = d4 * d4 + d9 * d9

                obj_term = 5.0 * coord + 2.0 * resp + irr + cls
                row = jnp.where(g4 > 0, obj_term, 0.5 * noobj)
                acc[...] += row

            @pl.loop(0, _HB, step=16)
            def _group(b0):
                do_group(b0)

        start(0, 0)

        @pl.loop(0, _SC_UNITS_PER_TILE - 1)
        def _unit(k):
            par = lax.rem(k, 2)
            start(k + 1, 1 - par)
            wait_work(k, par)

        wait_work(_SC_UNITS_PER_TILE - 1, (_SC_UNITS_PER_TILE - 1) % 2)

        pltpu.sync_copy(acc, out_hbm.at[wid])

    return body(pt, gt)


def _tc_loss_kernel(p_ref, g_ref, out_ref):
    i = pl.program_id(0)

    @pl.when(i == 0)
    def _init():
        out_ref[0, 0] = jnp.float32(0.0)

    p = p_ref[0]
    g = g_ref[0]

    def fc(a, c):
        return a[:, c, :]

    px1, py1, pw1, ph1, pc1 = (fc(p, c) for c in range(5))
    px2, py2, pw2, ph2, pc2 = (fc(p, c) for c in range(5, 10))
    gx, gy, gw, gh, g4 = (fc(g, c) for c in range(5))
    g9 = fc(g, 9)

    dcls = p[:, 10:, :] - g[:, 10:, :]
    cls = jnp.sum(dcls * dcls, axis=1)

    gltx = gx - _HALF * gw
    grbx = gx + _HALF * gw
    glty = gy - _HALF * gh
    grby = gy + _HALF * gh
    ag = (grbx - gltx) * (grby - glty)

    def iou_parts(px, py, pw, ph):
        pltx = px - _HALF * pw
        prbx = px + _HALF * pw
        plty = py - _HALF * ph
        prby = py + _HALF * ph
        wx = jnp.maximum(jnp.minimum(prbx, grbx) - jnp.maximum(pltx, gltx), 0.0)
        wy = jnp.maximum(jnp.minimum(prby, grby) - jnp.maximum(plty, glty), 0.0)
        inter = wx * wy
        ap = (prbx - pltx) * (prby - plty)
        return inter, ap + ag - inter + 1e-10

    in1, de1 = iou_parts(px1, py1, pw1, ph1)
    in2, de2 = iou_parts(px2, py2, pw2, ph2)
    sel = in2 * de1 > in1 * de2
    rx = jnp.where(sel, px2, px1)
    ry = jnp.where(sel, py2, py1)
    rw = jnp.where(sel, pw2, pw1)
    rh = jnp.where(sel, ph2, ph1)
    rc = jnp.where(sel, pc2, pc1)
    ic = jnp.where(sel, pc1, pc2)
    miou = jnp.where(sel, in2, in1) / jnp.where(sel, de2, de1)

    dx = rx - gx
    dy = ry - gy
    coord = (dx * dx + dy * dy
             + (rw + gw - 2.0 * jnp.sqrt(rw * gw))
             + (rh + gh - 2.0 * jnp.sqrt(rh * gh)))
    dresp = rc - miou
    resp = dresp * dresp
    irr = ic * ic
    d4 = pc1 - g4
    d9 = pc2 - g9
    noobj = d4 * d4 + d9 * d9

    obj_term = 5.0 * coord + 2.0 * resp + irr + cls
    row = jnp.where(g4 > 0, obj_term, 0.5 * noobj)
    out_ref[0, 0] += jnp.sum(row)


def _tc_loss(pt, gt):
    return pl.pallas_call(
        _tc_loss_kernel,
        grid=(_T_SPLIT,),
        in_specs=[
            pl.BlockSpec((1, _S, _C, _B), lambda i: (i, 0, 0, 0)),
            pl.BlockSpec((1, _S, _C, _B), lambda i: (i, 0, 0, 0)),
        ],
        out_specs=pl.BlockSpec(memory_space=pltpu.SMEM),
        out_shape=jax.ShapeDtypeStruct((1, 1), jnp.float32),
        compiler_params=pltpu.CompilerParams(
            dimension_semantics=("arbitrary",)),
    )(pt, gt)


def kernel(pred_tensor, ground_truth):
    pt = jnp.transpose(pred_tensor, (1, 2, 3, 0))
    gt = jnp.transpose(ground_truth, (1, 2, 3, 0))
    partials = _sc_partials(pt, gt)
    tc_part = _tc_loss(pt, gt)
    return (jnp.sum(partials) + tc_part[0, 0]) / _B


# speedup vs baseline: 1.0057x; 1.0057x over previous
import functools

import jax
import jax.numpy as jnp
from jax import lax
from jax.experimental import pallas as pl
from jax.experimental.pallas import tpu as pltpu
from jax.experimental.pallas import tpu_sc as plsc

_B = 512
_S = 14
_C = 30
_NTILES = 32
_HB = 256
_UPS = _B // _HB
_T_SPLIT = 6
_UNITS_TOTAL = _S * _S * _UPS
_SC_BASE = _T_SPLIT * _S * _UPS
_SC_UNITS = _UNITS_TOTAL - _SC_BASE
assert _SC_UNITS % _NTILES == 0
_SC_UNITS_PER_TILE = _SC_UNITS // _NTILES
_GROUPS = _HB // 16
_HALF = 0.5 * _S


def _sqrt16(x):
    i = plsc.bitcast(x, jnp.int32)
    i = jnp.int32(0x5F3759DF) - lax.shift_right_logical(i, 1)
    r = plsc.bitcast(i, jnp.float32)
    r = r * (1.5 - 0.5 * x * r * r)
    r = r * (1.5 - 0.5 * x * r * r)
    return x * r


def _sc_partials(pt, gt):
    mesh = plsc.VectorSubcoreMesh(core_axis_name="c", subcore_axis_name="s")

    @functools.partial(
        pl.kernel,
        mesh=mesh,
        out_type=jax.ShapeDtypeStruct((_NTILES, 16), jnp.float32),
        compiler_params=pltpu.CompilerParams(needs_layout_passes=False),
        scratch_types=[
            pltpu.VMEM((2, _C, _HB), jnp.float32),
            pltpu.VMEM((2, _C, _HB), jnp.float32),
            pltpu.VMEM((16,), jnp.float32),
            pltpu.SemaphoreType.DMA((2,)),
        ],
    )
    def body(p_hbm, g_hbm, out_hbm, pbufs, gbufs, acc, sems):
        wid = lax.axis_index("s") * 2 + lax.axis_index("c")
        acc[...] = jnp.zeros((16,), jnp.float32)

        def slices(uid):
            sid = uid // _UPS
            b0 = (uid - sid * _UPS) * _HB
            s1 = sid // _S
            s2 = sid - s1 * _S
            return (s1, s2, slice(None), pl.ds(b0, _HB))

        def start(k, par):
            uid = _SC_BASE + wid + k * _NTILES
            src = slices(uid)
            pltpu.async_copy(p_hbm.at[src], pbufs.at[par], sems.at[par])
            pltpu.async_copy(g_hbm.at[src], gbufs.at[par], sems.at[par])

        def wait_work(k, par):
            uid = _SC_BASE + wid + k * _NTILES
            src = slices(uid)
            pb = pbufs.at[par]
            gb = gbufs.at[par]
            pltpu.make_async_copy(p_hbm.at[src], pb, sems.at[par]).wait()
            pltpu.make_async_copy(g_hbm.at[src], gb, sems.at[par]).wait()

            def do_group(b0, a):
                def lp(c):
                    return pb[c, pl.ds(b0, 16)]

                def lg(c):
                    return gb[c, pl.ds(b0, 16)]

                px1, py1, pw1, ph1, pc1 = lp(0), lp(1), lp(2), lp(3), lp(4)
                px2, py2, pw2, ph2, pc2 = lp(5), lp(6), lp(7), lp(8), lp(9)
                gx, gy, gw, gh, g4 = lg(0), lg(1), lg(2), lg(3), lg(4)
                g9 = lg(9)

                cls = jnp.zeros((16,), jnp.float32)
                for c in range(10, _C):
                    dcv = lp(c) - lg(c)
                    cls = cls + dcv * dcv

                gltx = gx - _HALF * gw
                grbx = gx + _HALF * gw
                glty = gy - _HALF * gh
                grby = gy + _HALF * gh
                ag = (grbx - gltx) * (grby - glty)

                def iou_parts(px, py, pw, ph):
                    pltx = px - _HALF * pw
                    prbx = px + _HALF * pw
                    plty = py - _HALF * ph
                    prby = py + _HALF * ph
                    wx = jnp.maximum(
                        jnp.minimum(prbx, grbx) - jnp.maximum(pltx, gltx), 0.0)
                    wy = jnp.maximum(
                        jnp.minimum(prby, grby) - jnp.maximum(plty, glty), 0.0)
                    inter = wx * wy
                    ap = (prbx - pltx) * (prby - plty)
                    return inter, ap + ag - inter + 1e-10

                in1, de1 = iou_parts(px1, py1, pw1, ph1)
                in2, de2 = iou_parts(px2, py2, pw2, ph2)
                sel = in2 * de1 > in1 * de2
                rx = jnp.where(sel, px2, px1)
                ry = jnp.where(sel, py2, py1)
                rw = jnp.where(sel, pw2, pw1)
                rh = jnp.where(sel, ph2, ph1)
                rc = jnp.where(sel, pc2, pc1)
                ic = jnp.where(sel, pc1, pc2)
                miou = jnp.where(sel, in2, in1) / jnp.where(sel, de2, de1)

                dx = rx - gx
                dy = ry - gy
                coord = (dx * dx + dy * dy
                         + (rw + gw - 2.0 * _sqrt16(rw * gw))
                         + (rh + gh - 2.0 * _sqrt16(rh * gh)))
                dresp = rc - miou
                resp = dresp * dresp
                irr = ic * ic
                d4 = pc1 - g4
                d9 = pc2 - g9
                noobj = d4 * d4 + d9 * d9

                obj_term = 5.0 * coord + 2.0 * resp + irr + cls
                row = jnp.where(g4 > 0, obj_term, 0.5 * noobj)
                return a + row

            unit_sum = plsc.parallel_loop(
                0, _HB, step=16, unroll=2,
                carry=jnp.zeros((16,), jnp.float32))(do_group)
            acc[...] += unit_sum

        start(0, 0)

        @pl.loop(0, _SC_UNITS_PER_TILE - 1)
        def _unit(k):
            par = lax.rem(k, 2)
            start(k + 1, 1 - par)
            wait_work(k, par)

        wait_work(_SC_UNITS_PER_TILE - 1, (_SC_UNITS_PER_TILE - 1) % 2)

        pltpu.sync_copy(acc, out_hbm.at[wid])

    return body(pt, gt)


def _tc_loss_kernel(p_ref, g_ref, out_ref):
    i = pl.program_id(0)

    @pl.when(i == 0)
    def _init():
        out_ref[0, 0] = jnp.float32(0.0)

    p = p_ref[0]
    g = g_ref[0]

    def fc(a, c):
        return a[:, c, :]

    px1, py1, pw1, ph1, pc1 = (fc(p, c) for c in range(5))
    px2, py2, pw2, ph2, pc2 = (fc(p, c) for c in range(5, 10))
    gx, gy, gw, gh, g4 = (fc(g, c) for c in range(5))
    g9 = fc(g, 9)

    dcls = p[:, 10:, :] - g[:, 10:, :]
    cls = jnp.sum(dcls * dcls, axis=1)

    gltx = gx - _HALF * gw
    grbx = gx + _HALF * gw
    glty = gy - _HALF * gh
    grby = gy + _HALF * gh
    ag = (grbx - gltx) * (grby - glty)

    def iou_parts(px, py, pw, ph):
        pltx = px - _HALF * pw
        prbx = px + _HALF * pw
        plty = py - _HALF * ph
        prby = py + _HALF * ph
        wx = jnp.maximum(jnp.minimum(prbx, grbx) - jnp.maximum(pltx, gltx), 0.0)
        wy = jnp.maximum(jnp.minimum(prby, grby) - jnp.maximum(plty, glty), 0.0)
        inter = wx * wy
        ap = (prbx - pltx) * (prby - plty)
        return inter, ap + ag - inter + 1e-10

    in1, de1 = iou_parts(px1, py1, pw1, ph1)
    in2, de2 = iou_parts(px2, py2, pw2, ph2)
    sel = in2 * de1 > in1 * de2
    rx = jnp.where(sel, px2, px1)
    ry = jnp.where(sel, py2, py1)
    rw = jnp.where(sel, pw2, pw1)
    rh = jnp.where(sel, ph2, ph1)
    rc = jnp.where(sel, pc2, pc1)
    ic = jnp.where(sel, pc1, pc2)
    miou = jnp.where(sel, in2, in1) / jnp.where(sel, de2, de1)

    dx = rx - gx
    dy = ry - gy
    coord = (dx * dx + dy * dy
             + (rw + gw - 2.0 * jnp.sqrt(rw * gw))
             + (rh + gh - 2.0 * jnp.sqrt(rh * gh)))
    dresp = rc - miou
    resp = dresp * dresp
    irr = ic * ic
    d4 = pc1 - g4
    d9 = pc2 - g9
    noobj = d4 * d4 + d9 * d9

    obj_term = 5.0 * coord + 2.0 * resp + irr + cls
    row = jnp.where(g4 > 0, obj_term, 0.5 * noobj)
    out_ref[0, 0] += jnp.sum(row)


def _tc_loss(pt, gt):
    return pl.pallas_call(
        _tc_loss_kernel,
        grid=(_T_SPLIT,),
        in_specs=[
            pl.BlockSpec((1, _S, _C, _B), lambda i: (i, 0, 0, 0)),
            pl.BlockSpec((1, _S, _C, _B), lambda i: (i, 0, 0, 0)),
        ],
        out_specs=pl.BlockSpec(memory_space=pltpu.SMEM),
        out_shape=jax.ShapeDtypeStruct((1, 1), jnp.float32),
        compiler_params=pltpu.CompilerParams(
            dimension_semantics=("arbitrary",)),
    )(pt, gt)


def kernel(pred_tensor, ground_truth):
    pt = jnp.transpose(pred_tensor, (1, 2, 3, 0))
    gt = jnp.transpose(ground_truth, (1, 2, 3, 0))
    partials = _sc_partials(pt, gt)
    tc_part = _tc_loss(pt, gt)
    return (jnp.sum(partials) + tc_part[0, 0]) / _B
